# Initial kernel scaffold; baseline (speedup 1.0000x reference)
import jax
import jax.numpy as jnp
from jax.experimental import pallas as pl


def kernel(x, edge_index, w, W1, b1, W2, b2):
    raise NotImplementedError("write your pallas kernel here")


# speedup vs baseline: 2.9512x; 2.9512x over previous
import functools

import jax
import jax.numpy as jnp
from jax import lax
from jax.experimental import pallas as pl
from jax.experimental.pallas import tpu as pltpu
from jax.experimental.pallas import tpu_sc as plsc

N = 10000
E = 320000
D = 128

NC = 2
NS = 16
NW = NC * NS
K = 128
EPW = 10240
E2 = NW * EPW
NCH = EPW // K
SCH = 5
NSC = NCH // SCH
NP = 10240
RPT = NP // NS


def _sc_agg_body(with_cnt, *refs):
    (table, src3, dst3, w3, zeros,
     acc_out,
     src_v, dst_v, w_v, rows_v, acc_s, sem) = refs
    cidx = lax.axis_index("c")
    sidx = lax.axis_index("s")
    wid = cidx * NS + sidx

    pltpu.sync_copy(zeros.at[pl.ds(sidx * RPT, RPT)],
                    acc_s.at[pl.ds(sidx * RPT, RPT)])

    plsc.subcore_barrier()

    def super_body(sc, carry):
        pltpu.sync_copy(src3.at[wid, sc], src_v)
        pltpu.sync_copy(dst3.at[wid, sc], dst_v)
        pltpu.sync_copy(w3.at[wid, sc], w_v)

        def chunk_body(ch, c1):
            pltpu.async_copy(table.at[src_v.at[ch]], rows_v, sem).wait()

            def group_body(g, c2):
                w16 = w_v[ch, pl.ds(g * 16, 16)]
                for j in range(16):
                    e = g * 16 + j
                    wv = w16[j]
                    for r in range(D // 16):
                        sl = pl.ds(r * 16, 16)
                        rows_v[e, sl] = rows_v[e, sl] * wv
                return c2
            lax.fori_loop(0, K // 16, group_body, 0)

            pltpu.sync_copy(rows_v, acc_s.at[dst_v.at[ch]], add=True)
            return c1
        lax.fori_loop(0, SCH, chunk_body, 0)
        return carry
    lax.fori_loop(0, NSC, super_body, 0)

    plsc.subcore_barrier()

    pltpu.sync_copy(acc_s.at[pl.ds(sidx * RPT, RPT)],
                    acc_out.at[cidx, pl.ds(sidx * RPT, RPT)])


def _make_sc_agg():
    mesh = plsc.VectorSubcoreMesh(core_axis_name="c", subcore_axis_name="s")
    scratch = [
        pltpu.VMEM((SCH, K), jnp.int32),
        pltpu.VMEM((SCH, K), jnp.int32),
        pltpu.VMEM((SCH, K), jnp.float32),
        pltpu.VMEM((K, D), jnp.float32),
        pltpu.VMEM_SHARED((NP, D), jnp.float32),
        pltpu.SemaphoreType.DMA,
    ]
    return pl.kernel(
        functools.partial(_sc_agg_body, False),
        out_type=jax.ShapeDtypeStruct((NC, NP, D), jnp.float32),
        mesh=mesh,
        scratch_types=tuple(scratch),
    )


_sc_agg = _make_sc_agg()


def _sc_cnt_body(dst3, ones_h, zeros_c, cnt_out, dst_v, ones_v, cnt_s):
    cidx = lax.axis_index("c")
    sidx = lax.axis_index("s")
    wid = cidx * NS + sidx

    pltpu.sync_copy(zeros_c.at[pl.ds(sidx * RPT, RPT)],
                    cnt_s.at[pl.ds(sidx * RPT, RPT)])
    pltpu.sync_copy(ones_h, ones_v)
    plsc.subcore_barrier()

    def super_body(sc, carry):
        pltpu.sync_copy(dst3.at[wid, sc], dst_v)

        def chunk_body(ch, c1):
            pltpu.sync_copy(ones_v, cnt_s.at[dst_v.at[ch]], add=True)
            return c1
        lax.fori_loop(0, SCH, chunk_body, 0)
        return carry
    lax.fori_loop(0, NSC, super_body, 0)

    plsc.subcore_barrier()
    pltpu.sync_copy(cnt_s.at[pl.ds(sidx * RPT, RPT)],
                    cnt_out.at[cidx, pl.ds(sidx * RPT, RPT)])


_sc_cnt = pl.kernel(
    _sc_cnt_body,
    out_type=jax.ShapeDtypeStruct((NC, NP, D), jnp.float32),
    mesh=plsc.VectorSubcoreMesh(core_axis_name="c", subcore_axis_name="s"),
    scratch_types=(
        pltpu.VMEM((SCH, K), jnp.int32),
        pltpu.VMEM((K, D), jnp.float32),
        pltpu.VMEM_SHARED((NP, D), jnp.float32),
    ),
)

BT = 400
GT = N // BT


def _tc1_body(x_ref, acc_ref, cnt_ref, w1t_ref, b1_ref, w2t_ref, b2_ref,
              g_ref, s2_ref):
    cnt = cnt_ref[0, :, 0:1] + cnt_ref[1, :, 0:1]
    hN = (acc_ref[0] + acc_ref[1]) / jnp.maximum(cnt, 1.0)
    z = (jnp.dot(x_ref[...], w1t_ref[:D], preferred_element_type=jnp.float32)
         + jnp.dot(hN, w1t_ref[D:], preferred_element_type=jnp.float32)
         + b1_ref[...])
    h = jnp.maximum(z, 0.0)
    g_ref[...] = jnp.dot(h, w2t_ref[D:], preferred_element_type=jnp.float32)
    s2_ref[...] = (jnp.dot(h, w2t_ref[:D], preferred_element_type=jnp.float32)
                   + b2_ref[...])


_tc1 = pl.pallas_call(
    _tc1_body,
    grid=(GT,),
    in_specs=[
        pl.BlockSpec((BT, D), lambda i: (i, 0)),
        pl.BlockSpec((NC, BT, D), lambda i: (0, i, 0)),
        pl.BlockSpec((NC, BT, D), lambda i: (0, i, 0)),
        pl.BlockSpec((2 * D, D), lambda i: (0, 0)),
        pl.BlockSpec((1, D), lambda i: (0, 0)),
        pl.BlockSpec((2 * D, D), lambda i: (0, 0)),
        pl.BlockSpec((1, D), lambda i: (0, 0)),
    ],
    out_specs=[
        pl.BlockSpec((BT, D), lambda i: (i, 0)),
        pl.BlockSpec((BT, D), lambda i: (i, 0)),
    ],
    out_shape=[
        jax.ShapeDtypeStruct((N, D), jnp.float32),
        jax.ShapeDtypeStruct((N, D), jnp.float32),
    ],
)


def _tc2_body(s2_ref, acc_ref, cnt_ref, o_ref):
    cnt = cnt_ref[0, :, 0:1] + cnt_ref[1, :, 0:1]
    o_ref[...] = s2_ref[...] + (acc_ref[0] + acc_ref[1]) / jnp.maximum(cnt, 1.0)


_tc2 = pl.pallas_call(
    _tc2_body,
    grid=(GT,),
    in_specs=[
        pl.BlockSpec((BT, D), lambda i: (i, 0)),
        pl.BlockSpec((NC, BT, D), lambda i: (0, i, 0)),
        pl.BlockSpec((NC, BT, D), lambda i: (0, i, 0)),
    ],
    out_specs=pl.BlockSpec((BT, D), lambda i: (i, 0)),
    out_shape=jax.ShapeDtypeStruct((N, D), jnp.float32),
)


def kernel(x, edge_index, w, W1, b1, W2, b2):
    ei = edge_index.astype(jnp.int32)
    pad = E2 - E
    src_p = jnp.concatenate([ei[0], jnp.zeros((pad,), jnp.int32)])
    dst_p = jnp.concatenate([ei[1], jnp.full((pad,), N, jnp.int32)])
    w_p = jnp.concatenate([w.reshape(E).astype(jnp.float32),
                           jnp.zeros((pad,), jnp.float32)])
    src3 = src_p.reshape(NW, NSC, SCH, K)
    dst3 = dst_p.reshape(NW, NSC, SCH, K)
    w3 = w_p.reshape(NW, NSC, SCH, K)
    zeros = jnp.zeros((NP, D), jnp.float32)
    ones_h = jnp.ones((K, D), jnp.float32)
    W1T = W1.T
    W2T = W2.T
    b1r = b1.reshape(1, D)
    b2r = b2.reshape(1, D)

    cntp = _sc_cnt(dst3, ones_h, zeros)
    acc1 = _sc_agg(x, src3, dst3, w3, zeros)
    g, s2 = _tc1(x, acc1, cntp, W1T, b1r, W2T, b2r)
    acc2 = _sc_agg(g, src3, dst3, w3, zeros)
    return _tc2(s2, acc2, cntp)


# speedup vs baseline: 3.3454x; 1.1336x over previous
import functools

import jax
import jax.numpy as jnp
from jax import lax
from jax.experimental import pallas as pl
from jax.experimental.pallas import tpu as pltpu
from jax.experimental.pallas import tpu_sc as plsc

N = 10000
E = 320000
D = 128

NC = 2
NS = 16
NW = NC * NS
K = 128
EPW = 10240
E2 = NW * EPW
NCH = EPW // K
SCH = 4
NSC = NCH // SCH
NP = 10240
RPT = NP // NS


def _sc_agg_body(with_cnt, *refs):
    (table, src3, dst3, w3, zeros,
     acc_out,
     src_v, dst_v, w_v, rows_v, acc_s, sem0, sem1) = refs
    cidx = lax.axis_index("c")
    sidx = lax.axis_index("s")
    wid = cidx * NS + sidx
    sems = (sem0, sem1)

    def start_gather(sup, ch, b):
        pltpu.async_copy(table.at[src_v.at[ch]], rows_v.at[b], sems[b])

    def wait_gather(b):
        pltpu.make_async_copy(zeros.at[pl.ds(0, K)], rows_v.at[b],
                              sems[b]).wait()

    def stage(sup):
        pltpu.sync_copy(src3.at[wid, sup], src_v)
        pltpu.sync_copy(dst3.at[wid, sup], dst_v)
        pltpu.sync_copy(w3.at[wid, sup], w_v)

    def scale(ch, b):
        def group_body(g, c2):
            w16 = w_v[ch, pl.ds(g * 16, 16)]
            for j in range(16):
                e = g * 16 + j
                wv = w16[j]
                for r in range(D // 16):
                    sl = pl.ds(r * 16, 16)
                    rows_v[b, e, sl] = rows_v[b, e, sl] * wv
            return c2
        lax.fori_loop(0, K // 16, group_body, 0)

    def scatter(ch, b):
        pltpu.sync_copy(rows_v.at[b], acc_s.at[dst_v.at[ch]], add=True)

    pltpu.sync_copy(zeros.at[pl.ds(sidx * RPT, RPT)],
                    acc_s.at[pl.ds(sidx * RPT, RPT)])

    plsc.subcore_barrier()

    stage(0)
    start_gather(0, 0, 0)

    def super_body(sup, carry):
        for ch in range(SCH):
            b = ch % 2
            wait_gather(b)
            if ch < SCH - 1:
                start_gather(sup, ch + 1, 1 - b)
                scale(ch, b)
                scatter(ch, b)
            else:
                scale(ch, b)
                scatter(ch, b)

                @pl.when(sup != NSC - 1)
                def _():
                    stage(sup + 1)
                    start_gather(sup + 1, 0, 1 - b)
        return carry
    lax.fori_loop(0, NSC, super_body, 0)

    plsc.subcore_barrier()

    pltpu.sync_copy(acc_s.at[pl.ds(sidx * RPT, RPT)],
                    acc_out.at[cidx, pl.ds(sidx * RPT, RPT)])


def _make_sc_agg():
    mesh = plsc.VectorSubcoreMesh(core_axis_name="c", subcore_axis_name="s")
    scratch = [
        pltpu.VMEM((SCH, K), jnp.int32),
        pltpu.VMEM((SCH, K), jnp.int32),
        pltpu.VMEM((SCH, K), jnp.float32),
        pltpu.VMEM((2, K, D), jnp.float32),
        pltpu.VMEM_SHARED((NP, D), jnp.float32),
        pltpu.SemaphoreType.DMA,
        pltpu.SemaphoreType.DMA,
    ]
    return pl.kernel(
        functools.partial(_sc_agg_body, False),
        out_type=jax.ShapeDtypeStruct((NC, NP, D), jnp.float32),
        mesh=mesh,
        scratch_types=tuple(scratch),
    )


_sc_agg = _make_sc_agg()


def _sc_cnt_body(dst3, ones_h, zeros_c, cnt_out, dst_v, ones_v, cnt_s):
    cidx = lax.axis_index("c")
    sidx = lax.axis_index("s")
    wid = cidx * NS + sidx

    pltpu.sync_copy(zeros_c.at[pl.ds(sidx * RPT, RPT)],
                    cnt_s.at[pl.ds(sidx * RPT, RPT)])
    pltpu.sync_copy(ones_h, ones_v)
    plsc.subcore_barrier()

    def super_body(sc, carry):
        pltpu.sync_copy(dst3.at[wid, sc], dst_v)

        def chunk_body(ch, c1):
            pltpu.sync_copy(ones_v, cnt_s.at[dst_v.at[ch]], add=True)
            return c1
        lax.fori_loop(0, SCH, chunk_body, 0)
        return carry
    lax.fori_loop(0, NSC, super_body, 0)

    plsc.subcore_barrier()
    pltpu.sync_copy(cnt_s.at[pl.ds(sidx * RPT, RPT)],
                    cnt_out.at[cidx, pl.ds(sidx * RPT, RPT)])


_sc_cnt = pl.kernel(
    _sc_cnt_body,
    out_type=jax.ShapeDtypeStruct((NC, NP, D), jnp.float32),
    mesh=plsc.VectorSubcoreMesh(core_axis_name="c", subcore_axis_name="s"),
    scratch_types=(
        pltpu.VMEM((SCH, K), jnp.int32),
        pltpu.VMEM((K, D), jnp.float32),
        pltpu.VMEM_SHARED((NP, D), jnp.float32),
    ),
)

BT = 400
GT = N // BT


def _tc1_body(x_ref, acc_ref, cnt_ref, w1t_ref, b1_ref, w2t_ref, b2_ref,
              g_ref, s2_ref):
    cnt = cnt_ref[0, :, 0:1] + cnt_ref[1, :, 0:1]
    hN = (acc_ref[0] + acc_ref[1]) / jnp.maximum(cnt, 1.0)
    z = (jnp.dot(x_ref[...], w1t_ref[:D], preferred_element_type=jnp.float32)
         + jnp.dot(hN, w1t_ref[D:], preferred_element_type=jnp.float32)
         + b1_ref[...])
    h = jnp.maximum(z, 0.0)
    g_ref[...] = jnp.dot(h, w2t_ref[D:], preferred_element_type=jnp.float32)
    s2_ref[...] = (jnp.dot(h, w2t_ref[:D], preferred_element_type=jnp.float32)
                   + b2_ref[...])


_tc1 = pl.pallas_call(
    _tc1_body,
    grid=(GT,),
    in_specs=[
        pl.BlockSpec((BT, D), lambda i: (i, 0)),
        pl.BlockSpec((NC, BT, D), lambda i: (0, i, 0)),
        pl.BlockSpec((NC, BT, D), lambda i: (0, i, 0)),
        pl.BlockSpec((2 * D, D), lambda i: (0, 0)),
        pl.BlockSpec((1, D), lambda i: (0, 0)),
        pl.BlockSpec((2 * D, D), lambda i: (0, 0)),
        pl.BlockSpec((1, D), lambda i: (0, 0)),
    ],
    out_specs=[
        pl.BlockSpec((BT, D), lambda i: (i, 0)),
        pl.BlockSpec((BT, D), lambda i: (i, 0)),
    ],
    out_shape=[
        jax.ShapeDtypeStruct((N, D), jnp.float32),
        jax.ShapeDtypeStruct((N, D), jnp.float32),
    ],
)


def _tc2_body(s2_ref, acc_ref, cnt_ref, o_ref):
    cnt = cnt_ref[0, :, 0:1] + cnt_ref[1, :, 0:1]
    o_ref[...] = s2_ref[...] + (acc_ref[0] + acc_ref[1]) / jnp.maximum(cnt, 1.0)


_tc2 = pl.pallas_call(
    _tc2_body,
    grid=(GT,),
    in_specs=[
        pl.BlockSpec((BT, D), lambda i: (i, 0)),
        pl.BlockSpec((NC, BT, D), lambda i: (0, i, 0)),
        pl.BlockSpec((NC, BT, D), lambda i: (0, i, 0)),
    ],
    out_specs=pl.BlockSpec((BT, D), lambda i: (i, 0)),
    out_shape=jax.ShapeDtypeStruct((N, D), jnp.float32),
)


def kernel(x, edge_index, w, W1, b1, W2, b2):
    ei = edge_index.astype(jnp.int32)
    pad = E2 - E
    src_p = jnp.concatenate([ei[0], jnp.zeros((pad,), jnp.int32)])
    dst_p = jnp.concatenate([ei[1], jnp.full((pad,), N, jnp.int32)])
    w_p = jnp.concatenate([w.reshape(E).astype(jnp.float32),
                           jnp.zeros((pad,), jnp.float32)])
    src3 = src_p.reshape(NW, NSC, SCH, K)
    dst3 = dst_p.reshape(NW, NSC, SCH, K)
    w3 = w_p.reshape(NW, NSC, SCH, K)
    zeros = jnp.zeros((NP, D), jnp.float32)
    ones_h = jnp.ones((K, D), jnp.float32)
    W1T = W1.T
    W2T = W2.T
    b1r = b1.reshape(1, D)
    b2r = b2.reshape(1, D)

    cntp = _sc_cnt(dst3, ones_h, zeros)
    acc1 = _sc_agg(x, src3, dst3, w3, zeros)
    g, s2 = _tc1(x, acc1, cntp, W1T, b1r, W2T, b2r)
    acc2 = _sc_agg(g, src3, dst3, w3, zeros)
    return _tc2(s2, acc2, cntp)


# speedup vs baseline: 3.7700x; 1.1269x over previous
import functools

import jax
import jax.numpy as jnp
from jax import lax
from jax.experimental import pallas as pl
from jax.experimental.pallas import tpu as pltpu
from jax.experimental.pallas import tpu_sc as plsc

N = 10000
E = 320000
D = 128

NC = 2
NS = 16
NW = NC * NS
K = 128
EPW = 10240
E2 = NW * EPW
NCH = EPW // K
SCH = 4
NSC = NCH // SCH
TSUP = E2 // (SCH * K)
SUP0 = 28
SUP1 = 2 * NSC - SUP0
NP = 10240
RPT = NP // NS


def _sc_agg_body(with_cnt, *refs):
    (table, src3, dst3, w3, zeros,
     acc_out,
     src_v, dst_v, w_v, rows_v, acc_s, sem0, sem1) = refs
    cidx = lax.axis_index("c")
    sidx = lax.axis_index("s")
    base = jnp.where(cidx == 0, sidx * SUP0, NS * SUP0 + sidx * SUP1)
    nsup = jnp.where(cidx == 0, SUP0, SUP1)
    sems = (sem0, sem1)

    def start_gather(sup, ch, b):
        pltpu.async_copy(table.at[src_v.at[ch]], rows_v.at[b], sems[b])

    def wait_gather(b):
        pltpu.make_async_copy(zeros.at[pl.ds(0, K)], rows_v.at[b],
                              sems[b]).wait()

    def stage(sup):
        pltpu.sync_copy(src3.at[base + sup], src_v)
        pltpu.sync_copy(dst3.at[base + sup], dst_v)
        pltpu.sync_copy(w3.at[base + sup], w_v)

    def scale(ch, b):
        def group_body(g, c2):
            w16 = w_v[ch, pl.ds(g * 16, 16)]
            for j in range(16):
                e = g * 16 + j
                wv = w16[j]
                for r in range(D // 16):
                    sl = pl.ds(r * 16, 16)
                    rows_v[b, e, sl] = rows_v[b, e, sl] * wv
            return c2
        lax.fori_loop(0, K // 16, group_body, 0)

    def scatter(ch, b):
        pltpu.sync_copy(rows_v.at[b], acc_s.at[dst_v.at[ch]], add=True)

    pltpu.sync_copy(zeros.at[pl.ds(sidx * RPT, RPT)],
                    acc_s.at[pl.ds(sidx * RPT, RPT)])

    plsc.subcore_barrier()

    stage(0)
    start_gather(0, 0, 0)

    def super_body(sup, carry):
        for ch in range(SCH):
            b = ch % 2
            wait_gather(b)
            if ch < SCH - 1:
                start_gather(sup, ch + 1, 1 - b)
                scale(ch, b)
                scatter(ch, b)
            else:
                scale(ch, b)
                scatter(ch, b)

                @pl.when(sup != nsup - 1)
                def _():
                    stage(sup + 1)
                    start_gather(sup + 1, 0, 1 - b)
        return carry
    lax.fori_loop(0, nsup, super_body, 0)

    plsc.subcore_barrier()

    pltpu.sync_copy(acc_s.at[pl.ds(sidx * RPT, RPT)],
                    acc_out.at[cidx, pl.ds(sidx * RPT, RPT)])


def _make_sc_agg():
    mesh = plsc.VectorSubcoreMesh(core_axis_name="c", subcore_axis_name="s")
    scratch = [
        pltpu.VMEM((SCH, K), jnp.int32),
        pltpu.VMEM((SCH, K), jnp.int32),
        pltpu.VMEM((SCH, K), jnp.float32),
        pltpu.VMEM((2, K, D), jnp.float32),
        pltpu.VMEM_SHARED((NP, D), jnp.float32),
        pltpu.SemaphoreType.DMA,
        pltpu.SemaphoreType.DMA,
    ]
    return pl.kernel(
        functools.partial(_sc_agg_body, False),
        out_type=jax.ShapeDtypeStruct((NC, NP, D), jnp.float32),
        mesh=mesh,
        scratch_types=tuple(scratch),
    )


_sc_agg = _make_sc_agg()


def _sc_cnt_body(dst3, ones_h, zeros_c, cnt_out, dst_v, ones_v, cnt_s):
    cidx = lax.axis_index("c")
    sidx = lax.axis_index("s")
    wid = cidx * NS + sidx

    pltpu.sync_copy(zeros_c.at[pl.ds(sidx * RPT, RPT)],
                    cnt_s.at[pl.ds(sidx * RPT, RPT)])
    pltpu.sync_copy(ones_h, ones_v)
    plsc.subcore_barrier()

    def super_body(sc, carry):
        pltpu.sync_copy(dst3.at[wid * NSC + sc], dst_v)

        def chunk_body(ch, c1):
            pltpu.sync_copy(ones_v, cnt_s.at[dst_v.at[ch]], add=True)
            return c1
        lax.fori_loop(0, SCH, chunk_body, 0)
        return carry
    lax.fori_loop(0, NSC, super_body, 0)

    plsc.subcore_barrier()
    pltpu.sync_copy(cnt_s.at[pl.ds(sidx * RPT, RPT)],
                    cnt_out.at[cidx, pl.ds(sidx * RPT, RPT)])


_sc_cnt = pl.kernel(
    _sc_cnt_body,
    out_type=jax.ShapeDtypeStruct((NC, NP, D), jnp.float32),
    mesh=plsc.VectorSubcoreMesh(core_axis_name="c", subcore_axis_name="s"),
    scratch_types=(
        pltpu.VMEM((SCH, K), jnp.int32),
        pltpu.VMEM((K, D), jnp.float32),
        pltpu.VMEM_SHARED((NP, D), jnp.float32),
    ),
)

BT = 400
GT = N // BT


def _tc1_body(x_ref, acc_ref, cnt_ref, w1t_ref, b1_ref, w2t_ref, b2_ref,
              g_ref, s2_ref):
    cnt = cnt_ref[0, :, 0:1] + cnt_ref[1, :, 0:1]
    hN = (acc_ref[0] + acc_ref[1]) / jnp.maximum(cnt, 1.0)
    z = (jnp.dot(x_ref[...], w1t_ref[:D], preferred_element_type=jnp.float32)
         + jnp.dot(hN, w1t_ref[D:], preferred_element_type=jnp.float32)
         + b1_ref[...])
    h = jnp.maximum(z, 0.0)
    g_ref[...] = jnp.dot(h, w2t_ref[D:], preferred_element_type=jnp.float32)
    s2_ref[...] = (jnp.dot(h, w2t_ref[:D], preferred_element_type=jnp.float32)
                   + b2_ref[...])


_tc1 = pl.pallas_call(
    _tc1_body,
    grid=(GT,),
    in_specs=[
        pl.BlockSpec((BT, D), lambda i: (i, 0)),
        pl.BlockSpec((NC, BT, D), lambda i: (0, i, 0)),
        pl.BlockSpec((NC, BT, D), lambda i: (0, i, 0)),
        pl.BlockSpec((2 * D, D), lambda i: (0, 0)),
        pl.BlockSpec((1, D), lambda i: (0, 0)),
        pl.BlockSpec((2 * D, D), lambda i: (0, 0)),
        pl.BlockSpec((1, D), lambda i: (0, 0)),
    ],
    out_specs=[
        pl.BlockSpec((BT, D), lambda i: (i, 0)),
        pl.BlockSpec((BT, D), lambda i: (i, 0)),
    ],
    out_shape=[
        jax.ShapeDtypeStruct((N, D), jnp.float32),
        jax.ShapeDtypeStruct((N, D), jnp.float32),
    ],
)


def _tc2_body(s2_ref, acc_ref, cnt_ref, o_ref):
    cnt = cnt_ref[0, :, 0:1] + cnt_ref[1, :, 0:1]
    o_ref[...] = s2_ref[...] + (acc_ref[0] + acc_ref[1]) / jnp.maximum(cnt, 1.0)


_tc2 = pl.pallas_call(
    _tc2_body,
    grid=(GT,),
    in_specs=[
        pl.BlockSpec((BT, D), lambda i: (i, 0)),
        pl.BlockSpec((NC, BT, D), lambda i: (0, i, 0)),
        pl.BlockSpec((NC, BT, D), lambda i: (0, i, 0)),
    ],
    out_specs=pl.BlockSpec((BT, D), lambda i: (i, 0)),
    out_shape=jax.ShapeDtypeStruct((N, D), jnp.float32),
)


def kernel(x, edge_index, w, W1, b1, W2, b2):
    ei = edge_index.astype(jnp.int32)
    pad = E2 - E
    src_p = jnp.concatenate([ei[0], jnp.zeros((pad,), jnp.int32)])
    dst_p = jnp.concatenate([ei[1], jnp.full((pad,), N, jnp.int32)])
    w_p = jnp.concatenate([w.reshape(E).astype(jnp.float32),
                           jnp.zeros((pad,), jnp.float32)])
    src3 = src_p.reshape(TSUP, SCH, K)
    dst3 = dst_p.reshape(TSUP, SCH, K)
    w3 = w_p.reshape(TSUP, SCH, K)
    zeros = jnp.zeros((NP, D), jnp.float32)
    ones_h = jnp.ones((K, D), jnp.float32)
    W1T = W1.T
    W2T = W2.T
    b1r = b1.reshape(1, D)
    b2r = b2.reshape(1, D)

    cntp = _sc_cnt(dst3, ones_h, zeros)
    acc1 = _sc_agg(x, src3, dst3, w3, zeros)
    g, s2 = _tc1(x, acc1, cntp, W1T, b1r, W2T, b2r)
    acc2 = _sc_agg(g, src3, dst3, w3, zeros)
    return _tc2(s2, acc2, cntp)


# speedup vs baseline: 3.9003x; 1.0346x over previous
import functools

import jax
import jax.numpy as jnp
from jax import lax
from jax.experimental import pallas as pl
from jax.experimental.pallas import tpu as pltpu
from jax.experimental.pallas import tpu_sc as plsc

N = 10000
E = 320000
D = 128

NC = 2
NS = 16
NW = NC * NS
K = 128
EPW = 10240
E2 = NW * EPW
NCH = EPW // K
SCH = 4
NSC = NCH // SCH
TSUP = E2 // (SCH * K)
SUP0 = 30
SUP1 = 2 * NSC - SUP0
NP = 10240
RPT = NP // NS


def _sc_agg_body(with_cnt, *refs):
    (table, src3, dst3, w3, zeros,
     acc_out,
     src_v, dst_v, w_v, rows_v, acc_s, sem0, sem1) = refs
    cidx = lax.axis_index("c")
    sidx = lax.axis_index("s")
    base = jnp.where(cidx == 0, sidx * SUP0, NS * SUP0 + sidx * SUP1)
    nsup = jnp.where(cidx == 0, SUP0, SUP1)
    sems = (sem0, sem1)

    def start_gather(sup, ch, b):
        pltpu.async_copy(table.at[src_v.at[ch]], rows_v.at[b], sems[b])

    def wait_gather(b):
        pltpu.make_async_copy(zeros.at[pl.ds(0, K)], rows_v.at[b],
                              sems[b]).wait()

    def stage(sup):
        pltpu.sync_copy(src3.at[base + sup], src_v)
        pltpu.sync_copy(dst3.at[base + sup], dst_v)
        pltpu.sync_copy(w3.at[base + sup], w_v)

    def scale(ch, b):
        def group_body(g, c2):
            w16 = w_v[ch, pl.ds(g * 16, 16)]
            for j in range(16):
                e = g * 16 + j
                wv = w16[j]
                for r in range(D // 16):
                    sl = pl.ds(r * 16, 16)
                    rows_v[b, e, sl] = rows_v[b, e, sl] * wv
            return c2
        lax.fori_loop(0, K // 16, group_body, 0)

    def scatter(ch, b):
        pltpu.sync_copy(rows_v.at[b], acc_s.at[dst_v.at[ch]], add=True)

    pltpu.sync_copy(zeros.at[pl.ds(sidx * RPT, RPT)],
                    acc_s.at[pl.ds(sidx * RPT, RPT)])

    plsc.subcore_barrier()

    stage(0)
    start_gather(0, 0, 0)

    def super_body(sup, carry):
        for ch in range(SCH):
            b = ch % 2
            wait_gather(b)
            if ch < SCH - 1:
                start_gather(sup, ch + 1, 1 - b)
                scale(ch, b)
                scatter(ch, b)
            else:
                scale(ch, b)
                scatter(ch, b)

                @pl.when(sup != nsup - 1)
                def _():
                    stage(sup + 1)
                    start_gather(sup + 1, 0, 1 - b)
        return carry
    lax.fori_loop(0, nsup, super_body, 0)

    plsc.subcore_barrier()

    pltpu.sync_copy(acc_s.at[pl.ds(sidx * RPT, RPT)],
                    acc_out.at[cidx, pl.ds(sidx * RPT, RPT)])


def _make_sc_agg():
    mesh = plsc.VectorSubcoreMesh(core_axis_name="c", subcore_axis_name="s")
    scratch = [
        pltpu.VMEM((SCH, K), jnp.int32),
        pltpu.VMEM((SCH, K), jnp.int32),
        pltpu.VMEM((SCH, K), jnp.float32),
        pltpu.VMEM((2, K, D), jnp.float32),
        pltpu.VMEM_SHARED((NP, D), jnp.float32),
        pltpu.SemaphoreType.DMA,
        pltpu.SemaphoreType.DMA,
    ]
    return pl.kernel(
        functools.partial(_sc_agg_body, False),
        out_type=jax.ShapeDtypeStruct((NC, NP, D), jnp.float32),
        mesh=mesh,
        scratch_types=tuple(scratch),
    )


_sc_agg = _make_sc_agg()


def _sc_cnt_body(dst3, ones_h, zeros_c, cnt_out, dst_v, ones_v, cnt_s):
    cidx = lax.axis_index("c")
    sidx = lax.axis_index("s")
    wid = cidx * NS + sidx

    pltpu.sync_copy(zeros_c.at[pl.ds(sidx * RPT, RPT)],
                    cnt_s.at[pl.ds(sidx * RPT, RPT)])
    pltpu.sync_copy(ones_h, ones_v)
    plsc.subcore_barrier()

    def super_body(sc, carry):
        pltpu.sync_copy(dst3.at[wid * NSC + sc], dst_v)

        def chunk_body(ch, c1):
            pltpu.sync_copy(ones_v, cnt_s.at[dst_v.at[ch]], add=True)
            return c1
        lax.fori_loop(0, SCH, chunk_body, 0)
        return carry
    lax.fori_loop(0, NSC, super_body, 0)

    plsc.subcore_barrier()
    pltpu.sync_copy(cnt_s.at[pl.ds(sidx * RPT, RPT)],
                    cnt_out.at[cidx, pl.ds(sidx * RPT, RPT)])


_sc_cnt = pl.kernel(
    _sc_cnt_body,
    out_type=jax.ShapeDtypeStruct((NC, NP, D), jnp.float32),
    mesh=plsc.VectorSubcoreMesh(core_axis_name="c", subcore_axis_name="s"),
    scratch_types=(
        pltpu.VMEM((SCH, K), jnp.int32),
        pltpu.VMEM((K, D), jnp.float32),
        pltpu.VMEM_SHARED((NP, D), jnp.float32),
    ),
)

BT = 400
GT = N // BT


def _tc1_body(x_ref, acc_ref, cnt_ref, w1t_ref, b1_ref, w2t_ref, b2_ref,
              g_ref, s2_ref):
    cnt = cnt_ref[0, :, 0:1] + cnt_ref[1, :, 0:1]
    hN = (acc_ref[0] + acc_ref[1]) / jnp.maximum(cnt, 1.0)
    z = (jnp.dot(x_ref[...], w1t_ref[:D], preferred_element_type=jnp.float32)
         + jnp.dot(hN, w1t_ref[D:], preferred_element_type=jnp.float32)
         + b1_ref[...])
    h = jnp.maximum(z, 0.0)
    g_ref[...] = jnp.dot(h, w2t_ref[D:], preferred_element_type=jnp.float32)
    s2_ref[...] = (jnp.dot(h, w2t_ref[:D], preferred_element_type=jnp.float32)
                   + b2_ref[...])


_tc1 = pl.pallas_call(
    _tc1_body,
    grid=(GT,),
    in_specs=[
        pl.BlockSpec((BT, D), lambda i: (i, 0)),
        pl.BlockSpec((NC, BT, D), lambda i: (0, i, 0)),
        pl.BlockSpec((NC, BT, D), lambda i: (0, i, 0)),
        pl.BlockSpec((2 * D, D), lambda i: (0, 0)),
        pl.BlockSpec((1, D), lambda i: (0, 0)),
        pl.BlockSpec((2 * D, D), lambda i: (0, 0)),
        pl.BlockSpec((1, D), lambda i: (0, 0)),
    ],
    out_specs=[
        pl.BlockSpec((BT, D), lambda i: (i, 0)),
        pl.BlockSpec((BT, D), lambda i: (i, 0)),
    ],
    out_shape=[
        jax.ShapeDtypeStruct((N, D), jnp.float32),
        jax.ShapeDtypeStruct((N, D), jnp.float32),
    ],
)


def _tc2_body(s2_ref, acc_ref, cnt_ref, o_ref):
    cnt = cnt_ref[0, :, 0:1] + cnt_ref[1, :, 0:1]
    o_ref[...] = s2_ref[...] + (acc_ref[0] + acc_ref[1]) / jnp.maximum(cnt, 1.0)


_tc2 = pl.pallas_call(
    _tc2_body,
    grid=(GT,),
    in_specs=[
        pl.BlockSpec((BT, D), lambda i: (i, 0)),
        pl.BlockSpec((NC, BT, D), lambda i: (0, i, 0)),
        pl.BlockSpec((NC, BT, D), lambda i: (0, i, 0)),
    ],
    out_specs=pl.BlockSpec((BT, D), lambda i: (i, 0)),
    out_shape=jax.ShapeDtypeStruct((N, D), jnp.float32),
)


def kernel(x, edge_index, w, W1, b1, W2, b2):
    ei = edge_index.astype(jnp.int32)
    pad = E2 - E
    src_p = jnp.concatenate([ei[0], jnp.zeros((pad,), jnp.int32)])
    dst_p = jnp.concatenate([ei[1], jnp.full((pad,), N, jnp.int32)])
    w_p = jnp.concatenate([w.reshape(E).astype(jnp.float32),
                           jnp.zeros((pad,), jnp.float32)])
    src3 = src_p.reshape(TSUP, SCH, K)
    dst3 = dst_p.reshape(TSUP, SCH, K)
    w3 = w_p.reshape(TSUP, SCH, K)
    zeros = jnp.zeros((NP, D), jnp.float32)
    ones_h = jnp.ones((K, D), jnp.float32)
    W1T = W1.T
    W2T = W2.T
    b1r = b1.reshape(1, D)
    b2r = b2.reshape(1, D)

    cntp = _sc_cnt(dst3, ones_h, zeros)
    acc1 = _sc_agg(x, src3, dst3, w3, zeros)
    g, s2 = _tc1(x, acc1, cntp, W1T, b1r, W2T, b2r)
    acc2 = _sc_agg(g, src3, dst3, w3, zeros)
    return _tc2(s2, acc2, cntp)


# speedup vs baseline: 4.0809x; 1.0463x over previous
import functools

import jax
import jax.numpy as jnp
from jax import lax
from jax.experimental import pallas as pl
from jax.experimental.pallas import tpu as pltpu
from jax.experimental.pallas import tpu_sc as plsc

N = 10000
E = 320000
D = 128

NC = 2
NS = 16
NW = NC * NS
K = 128
EPW = 10240
E2 = NW * EPW
NCH = EPW // K
SCH = 4
NSC = NCH // SCH
TSUP = E2 // (SCH * K)
SUP0 = 33
SUP1 = 2 * NSC - SUP0
NP = 10240
RPT = NP // NS


def _sc_agg_body(with_cnt, *refs):
    (table, src3, dst3, w3, zeros,
     acc_out,
     src_v, dst_v, w_v, rows_v, acc_s, sem0, sem1) = refs
    cidx = lax.axis_index("c")
    sidx = lax.axis_index("s")
    base = jnp.where(cidx == 0, sidx * SUP0, NS * SUP0 + sidx * SUP1)
    nsup = jnp.where(cidx == 0, SUP0, SUP1)
    sems = (sem0, sem1)

    def start_gather(sup, ch, b):
        pltpu.async_copy(table.at[src_v.at[ch]], rows_v.at[b], sems[b])

    def wait_gather(b):
        pltpu.make_async_copy(zeros.at[pl.ds(0, K)], rows_v.at[b],
                              sems[b]).wait()

    def stage(sup):
        pltpu.sync_copy(src3.at[base + sup], src_v)
        pltpu.sync_copy(dst3.at[base + sup], dst_v)
        pltpu.sync_copy(w3.at[base + sup], w_v)

    def scale(ch, b):
        def group_body(g, c2):
            w16 = w_v[ch, pl.ds(g * 16, 16)]
            for j in range(16):
                e = g * 16 + j
                wv = w16[j]
                for r in range(D // 16):
                    sl = pl.ds(r * 16, 16)
                    rows_v[b, e, sl] = rows_v[b, e, sl] * wv
            return c2
        lax.fori_loop(0, K // 16, group_body, 0)

    def scatter(ch, b):
        pltpu.sync_copy(rows_v.at[b], acc_s.at[dst_v.at[ch]], add=True)

    pltpu.sync_copy(zeros.at[pl.ds(sidx * RPT, RPT)],
                    acc_s.at[pl.ds(sidx * RPT, RPT)])

    plsc.subcore_barrier()

    stage(0)
    start_gather(0, 0, 0)

    def super_body(sup, carry):
        for ch in range(SCH):
            b = ch % 2
            wait_gather(b)
            if ch < SCH - 1:
                start_gather(sup, ch + 1, 1 - b)
                scale(ch, b)
                scatter(ch, b)
            else:
                scale(ch, b)
                scatter(ch, b)

                @pl.when(sup != nsup - 1)
                def _():
                    stage(sup + 1)
                    start_gather(sup + 1, 0, 1 - b)
        return carry
    lax.fori_loop(0, nsup, super_body, 0)

    plsc.subcore_barrier()

    pltpu.sync_copy(acc_s.at[pl.ds(sidx * RPT, RPT)],
                    acc_out.at[cidx, pl.ds(sidx * RPT, RPT)])


def _make_sc_agg():
    mesh = plsc.VectorSubcoreMesh(core_axis_name="c", subcore_axis_name="s")
    scratch = [
        pltpu.VMEM((SCH, K), jnp.int32),
        pltpu.VMEM((SCH, K), jnp.int32),
        pltpu.VMEM((SCH, K), jnp.float32),
        pltpu.VMEM((2, K, D), jnp.float32),
        pltpu.VMEM_SHARED((NP, D), jnp.float32),
        pltpu.SemaphoreType.DMA,
        pltpu.SemaphoreType.DMA,
    ]
    return pl.kernel(
        functools.partial(_sc_agg_body, False),
        out_type=jax.ShapeDtypeStruct((NC, NP, D), jnp.float32),
        mesh=mesh,
        scratch_types=tuple(scratch),
    )


_sc_agg = _make_sc_agg()


def _sc_cnt_body(dst3, ones_h, zeros_c, cnt_out, dst_v, ones_v, cnt_s):
    cidx = lax.axis_index("c")
    sidx = lax.axis_index("s")
    wid = cidx * NS + sidx

    pltpu.sync_copy(zeros_c.at[pl.ds(sidx * RPT, RPT)],
                    cnt_s.at[pl.ds(sidx * RPT, RPT)])
    pltpu.sync_copy(ones_h, ones_v)
    plsc.subcore_barrier()

    def super_body(sc, carry):
        pltpu.sync_copy(dst3.at[wid * NSC + sc], dst_v)

        def chunk_body(ch, c1):
            pltpu.sync_copy(ones_v, cnt_s.at[dst_v.at[ch]], add=True)
            return c1
        lax.fori_loop(0, SCH, chunk_body, 0)
        return carry
    lax.fori_loop(0, NSC, super_body, 0)

    plsc.subcore_barrier()
    pltpu.sync_copy(cnt_s.at[pl.ds(sidx * RPT, RPT)],
                    cnt_out.at[cidx, pl.ds(sidx * RPT, RPT)])


_sc_cnt = pl.kernel(
    _sc_cnt_body,
    out_type=jax.ShapeDtypeStruct((NC, NP, D), jnp.float32),
    mesh=plsc.VectorSubcoreMesh(core_axis_name="c", subcore_axis_name="s"),
    scratch_types=(
        pltpu.VMEM((SCH, K), jnp.int32),
        pltpu.VMEM((K, D), jnp.float32),
        pltpu.VMEM_SHARED((NP, D), jnp.float32),
    ),
)

BT = 400
GT = N // BT


def _tc1_body(x_ref, acc_ref, cnt_ref, w1t_ref, b1_ref, w2t_ref, b2_ref,
              g_ref, s2_ref):
    cnt = cnt_ref[0, :, 0:1] + cnt_ref[1, :, 0:1]
    hN = (acc_ref[0] + acc_ref[1]) / jnp.maximum(cnt, 1.0)
    z = (jnp.dot(x_ref[...], w1t_ref[:D], preferred_element_type=jnp.float32)
         + jnp.dot(hN, w1t_ref[D:], preferred_element_type=jnp.float32)
         + b1_ref[...])
    h = jnp.maximum(z, 0.0)
    g_ref[...] = jnp.dot(h, w2t_ref[D:], preferred_element_type=jnp.float32)
    s2_ref[...] = (jnp.dot(h, w2t_ref[:D], preferred_element_type=jnp.float32)
                   + b2_ref[...])


_tc1 = pl.pallas_call(
    _tc1_body,
    grid=(GT,),
    in_specs=[
        pl.BlockSpec((BT, D), lambda i: (i, 0)),
        pl.BlockSpec((NC, BT, D), lambda i: (0, i, 0)),
        pl.BlockSpec((NC, BT, D), lambda i: (0, i, 0)),
        pl.BlockSpec((2 * D, D), lambda i: (0, 0)),
        pl.BlockSpec((1, D), lambda i: (0, 0)),
        pl.BlockSpec((2 * D, D), lambda i: (0, 0)),
        pl.BlockSpec((1, D), lambda i: (0, 0)),
    ],
    out_specs=[
        pl.BlockSpec((BT, D), lambda i: (i, 0)),
        pl.BlockSpec((BT, D), lambda i: (i, 0)),
    ],
    out_shape=[
        jax.ShapeDtypeStruct((N, D), jnp.float32),
        jax.ShapeDtypeStruct((N, D), jnp.float32),
    ],
)


def _tc2_body(s2_ref, acc_ref, cnt_ref, o_ref):
    cnt = cnt_ref[0, :, 0:1] + cnt_ref[1, :, 0:1]
    o_ref[...] = s2_ref[...] + (acc_ref[0] + acc_ref[1]) / jnp.maximum(cnt, 1.0)


_tc2 = pl.pallas_call(
    _tc2_body,
    grid=(GT,),
    in_specs=[
        pl.BlockSpec((BT, D), lambda i: (i, 0)),
        pl.BlockSpec((NC, BT, D), lambda i: (0, i, 0)),
        pl.BlockSpec((NC, BT, D), lambda i: (0, i, 0)),
    ],
    out_specs=pl.BlockSpec((BT, D), lambda i: (i, 0)),
    out_shape=jax.ShapeDtypeStruct((N, D), jnp.float32),
)


def kernel(x, edge_index, w, W1, b1, W2, b2):
    ei = edge_index.astype(jnp.int32)
    pad = E2 - E
    src_p = jnp.concatenate([ei[0], jnp.zeros((pad,), jnp.int32)])
    dst_p = jnp.concatenate([ei[1], jnp.full((pad,), N, jnp.int32)])
    w_p = jnp.concatenate([w.reshape(E).astype(jnp.float32),
                           jnp.zeros((pad,), jnp.float32)])
    src3 = src_p.reshape(TSUP, SCH, K)
    dst3 = dst_p.reshape(TSUP, SCH, K)
    w3 = w_p.reshape(TSUP, SCH, K)
    zeros = jnp.zeros((NP, D), jnp.float32)
    ones_h = jnp.ones((K, D), jnp.float32)
    W1T = W1.T
    W2T = W2.T
    b1r = b1.reshape(1, D)
    b2r = b2.reshape(1, D)

    cntp = _sc_cnt(dst3, ones_h, zeros)
    acc1 = _sc_agg(x, src3, dst3, w3, zeros)
    g, s2 = _tc1(x, acc1, cntp, W1T, b1r, W2T, b2r)
    acc2 = _sc_agg(g, src3, dst3, w3, zeros)
    return _tc2(s2, acc2, cntp)


# speedup vs baseline: 4.1466x; 1.0161x over previous
import functools

import jax
import jax.numpy as jnp
from jax import lax
from jax.experimental import pallas as pl
from jax.experimental.pallas import tpu as pltpu
from jax.experimental.pallas import tpu_sc as plsc

N = 10000
E = 320000
D = 128

NC = 2
NS = 16
NW = NC * NS
K = 128
EPW = 10240
E2 = NW * EPW
NCH = EPW // K
SCH = 4
NSC = NCH // SCH
TSUP = E2 // (SCH * K)
SUP0 = 36
SUP1 = 2 * NSC - SUP0
NP = 10240
RPT = NP // NS


def _sc_agg_body(with_cnt, *refs):
    (table, src3, dst3, w3, zeros,
     acc_out,
     src_v, dst_v, w_v, rows_v, acc_s, sem0, sem1) = refs
    cidx = lax.axis_index("c")
    sidx = lax.axis_index("s")
    base = jnp.where(cidx == 0, sidx * SUP0, NS * SUP0 + sidx * SUP1)
    nsup = jnp.where(cidx == 0, SUP0, SUP1)
    sems = (sem0, sem1)

    def start_gather(sup, ch, b):
        pltpu.async_copy(table.at[src_v.at[ch]], rows_v.at[b], sems[b])

    def wait_gather(b):
        pltpu.make_async_copy(zeros.at[pl.ds(0, K)], rows_v.at[b],
                              sems[b]).wait()

    def stage(sup):
        pltpu.sync_copy(src3.at[base + sup], src_v)
        pltpu.sync_copy(dst3.at[base + sup], dst_v)
        pltpu.sync_copy(w3.at[base + sup], w_v)

    def scale(ch, b):
        def group_body(g, c2):
            w16 = w_v[ch, pl.ds(g * 16, 16)]
            for j in range(16):
                e = g * 16 + j
                wv = w16[j]
                for r in range(D // 16):
                    sl = pl.ds(r * 16, 16)
                    rows_v[b, e, sl] = rows_v[b, e, sl] * wv
            return c2
        lax.fori_loop(0, K // 16, group_body, 0)

    def scatter(ch, b):
        pltpu.sync_copy(rows_v.at[b], acc_s.at[dst_v.at[ch]], add=True)

    pltpu.sync_copy(zeros.at[pl.ds(sidx * RPT, RPT)],
                    acc_s.at[pl.ds(sidx * RPT, RPT)])

    plsc.subcore_barrier()

    stage(0)
    start_gather(0, 0, 0)

    def super_body(sup, carry):
        for ch in range(SCH):
            b = ch % 2
            wait_gather(b)
            if ch < SCH - 1:
                start_gather(sup, ch + 1, 1 - b)
                scale(ch, b)
                scatter(ch, b)
            else:
                scale(ch, b)
                scatter(ch, b)

                @pl.when(sup != nsup - 1)
                def _():
                    stage(sup + 1)
                    start_gather(sup + 1, 0, 1 - b)
        return carry
    lax.fori_loop(0, nsup, super_body, 0)

    plsc.subcore_barrier()

    pltpu.sync_copy(acc_s.at[pl.ds(sidx * RPT, RPT)],
                    acc_out.at[cidx, pl.ds(sidx * RPT, RPT)])


def _make_sc_agg():
    mesh = plsc.VectorSubcoreMesh(core_axis_name="c", subcore_axis_name="s")
    scratch = [
        pltpu.VMEM((SCH, K), jnp.int32),
        pltpu.VMEM((SCH, K), jnp.int32),
        pltpu.VMEM((SCH, K), jnp.float32),
        pltpu.VMEM((2, K, D), jnp.float32),
        pltpu.VMEM_SHARED((NP, D), jnp.float32),
        pltpu.SemaphoreType.DMA,
        pltpu.SemaphoreType.DMA,
    ]
    return pl.kernel(
        functools.partial(_sc_agg_body, False),
        out_type=jax.ShapeDtypeStruct((NC, NP, D), jnp.float32),
        mesh=mesh,
        scratch_types=tuple(scratch),
    )


_sc_agg = _make_sc_agg()


def _sc_cnt_body(dst3, ones_h, zeros_c, cnt_out, dst_v, ones_v, cnt_s):
    cidx = lax.axis_index("c")
    sidx = lax.axis_index("s")
    wid = cidx * NS + sidx

    pltpu.sync_copy(zeros_c.at[pl.ds(sidx * RPT, RPT)],
                    cnt_s.at[pl.ds(sidx * RPT, RPT)])
    pltpu.sync_copy(ones_h, ones_v)
    plsc.subcore_barrier()

    def super_body(sc, carry):
        pltpu.sync_copy(dst3.at[wid * NSC + sc], dst_v)

        def chunk_body(ch, c1):
            pltpu.sync_copy(ones_v, cnt_s.at[dst_v.at[ch]], add=True)
            return c1
        lax.fori_loop(0, SCH, chunk_body, 0)
        return carry
    lax.fori_loop(0, NSC, super_body, 0)

    plsc.subcore_barrier()
    pltpu.sync_copy(cnt_s.at[pl.ds(sidx * RPT, RPT)],
                    cnt_out.at[cidx, pl.ds(sidx * RPT, RPT)])


_sc_cnt = pl.kernel(
    _sc_cnt_body,
    out_type=jax.ShapeDtypeStruct((NC, NP, D), jnp.float32),
    mesh=plsc.VectorSubcoreMesh(core_axis_name="c", subcore_axis_name="s"),
    scratch_types=(
        pltpu.VMEM((SCH, K), jnp.int32),
        pltpu.VMEM((K, D), jnp.float32),
        pltpu.VMEM_SHARED((NP, D), jnp.float32),
    ),
)

BT = 400
GT = N // BT


def _tc1_body(x_ref, acc_ref, cnt_ref, w1t_ref, b1_ref, w2t_ref, b2_ref,
              g_ref, s2_ref):
    cnt = cnt_ref[0, :, 0:1] + cnt_ref[1, :, 0:1]
    hN = (acc_ref[0] + acc_ref[1]) / jnp.maximum(cnt, 1.0)
    z = (jnp.dot(x_ref[...], w1t_ref[:D], preferred_element_type=jnp.float32)
         + jnp.dot(hN, w1t_ref[D:], preferred_element_type=jnp.float32)
         + b1_ref[...])
    h = jnp.maximum(z, 0.0)
    g_ref[...] = jnp.dot(h, w2t_ref[D:], preferred_element_type=jnp.float32)
    s2_ref[...] = (jnp.dot(h, w2t_ref[:D], preferred_element_type=jnp.float32)
                   + b2_ref[...])


_tc1 = pl.pallas_call(
    _tc1_body,
    grid=(GT,),
    in_specs=[
        pl.BlockSpec((BT, D), lambda i: (i, 0)),
        pl.BlockSpec((NC, BT, D), lambda i: (0, i, 0)),
        pl.BlockSpec((NC, BT, D), lambda i: (0, i, 0)),
        pl.BlockSpec((2 * D, D), lambda i: (0, 0)),
        pl.BlockSpec((1, D), lambda i: (0, 0)),
        pl.BlockSpec((2 * D, D), lambda i: (0, 0)),
        pl.BlockSpec((1, D), lambda i: (0, 0)),
    ],
    out_specs=[
        pl.BlockSpec((BT, D), lambda i: (i, 0)),
        pl.BlockSpec((BT, D), lambda i: (i, 0)),
    ],
    out_shape=[
        jax.ShapeDtypeStruct((N, D), jnp.float32),
        jax.ShapeDtypeStruct((N, D), jnp.float32),
    ],
)


def _tc2_body(s2_ref, acc_ref, cnt_ref, o_ref):
    cnt = cnt_ref[0, :, 0:1] + cnt_ref[1, :, 0:1]
    o_ref[...] = s2_ref[...] + (acc_ref[0] + acc_ref[1]) / jnp.maximum(cnt, 1.0)


_tc2 = pl.pallas_call(
    _tc2_body,
    grid=(GT,),
    in_specs=[
        pl.BlockSpec((BT, D), lambda i: (i, 0)),
        pl.BlockSpec((NC, BT, D), lambda i: (0, i, 0)),
        pl.BlockSpec((NC, BT, D), lambda i: (0, i, 0)),
    ],
    out_specs=pl.BlockSpec((BT, D), lambda i: (i, 0)),
    out_shape=jax.ShapeDtypeStruct((N, D), jnp.float32),
)


def kernel(x, edge_index, w, W1, b1, W2, b2):
    ei = edge_index.astype(jnp.int32)
    pad = E2 - E
    src_p = jnp.concatenate([ei[0], jnp.zeros((pad,), jnp.int32)])
    dst_p = jnp.concatenate([ei[1], jnp.full((pad,), N, jnp.int32)])
    w_p = jnp.concatenate([w.reshape(E).astype(jnp.float32),
                           jnp.zeros((pad,), jnp.float32)])
    src3 = src_p.reshape(TSUP, SCH, K)
    dst3 = dst_p.reshape(TSUP, SCH, K)
    w3 = w_p.reshape(TSUP, SCH, K)
    zeros = jnp.zeros((NP, D), jnp.float32)
    ones_h = jnp.ones((K, D), jnp.float32)
    W1T = W1.T
    W2T = W2.T
    b1r = b1.reshape(1, D)
    b2r = b2.reshape(1, D)

    cntp = _sc_cnt(dst3, ones_h, zeros)
    acc1 = _sc_agg(x, src3, dst3, w3, zeros)
    g, s2 = _tc1(x, acc1, cntp, W1T, b1r, W2T, b2r)
    acc2 = _sc_agg(g, src3, dst3, w3, zeros)
    return _tc2(s2, acc2, cntp)


# speedup vs baseline: 4.2830x; 1.0329x over previous
import functools

import jax
import jax.numpy as jnp
from jax import lax
from jax.experimental import pallas as pl
from jax.experimental.pallas import tpu as pltpu
from jax.experimental.pallas import tpu_sc as plsc

N = 10000
E = 320000
D = 128

NC = 2
NS = 16
NW = NC * NS
K = 128
EPW = 10240
E2 = NW * EPW
NCH = EPW // K
SCH = 4
NSC = NCH // SCH
TSUP = E2 // (SCH * K)
SUP0 = 36
SUP1 = 2 * NSC - SUP0
NP = 10240
RPT = NP // NS


def _sc_agg_body(with_cnt, *refs):
    (table, src3, dst3, w3, zeros,
     acc_out,
     src_v, dst_v, w_v, rows_v, acc_s, sem0, sem1, ssem0, ssem1) = refs
    cidx = lax.axis_index("c")
    sidx = lax.axis_index("s")
    base = jnp.where(cidx == 0, sidx * SUP0, NS * SUP0 + sidx * SUP1)
    nsup = jnp.where(cidx == 0, SUP0, SUP1)
    sems = (sem0, sem1)
    ssems = (ssem0, ssem1)

    def start_gather(sup, ch, b):
        pltpu.async_copy(table.at[src_v.at[ch]], rows_v.at[b], sems[b])

    def wait_gather(b):
        pltpu.make_async_copy(zeros.at[pl.ds(0, K)], rows_v.at[b],
                              sems[b]).wait()

    def stage(sup):
        pltpu.sync_copy(src3.at[base + sup], src_v)
        pltpu.sync_copy(dst3.at[base + sup], dst_v)
        pltpu.sync_copy(w3.at[base + sup], w_v)

    def scale(ch, b):
        def group_body(g, c2):
            w16 = w_v[ch, pl.ds(g * 16, 16)]
            for j in range(16):
                e = g * 16 + j
                wv = w16[j]
                for r in range(D // 16):
                    sl = pl.ds(r * 16, 16)
                    rows_v[b, e, sl] = rows_v[b, e, sl] * wv
            return c2
        lax.fori_loop(0, K // 16, group_body, 0)

    def start_scatter(ch, b):
        pltpu.async_copy(rows_v.at[b], acc_s.at[dst_v.at[ch]], ssems[b],
                         add=True)

    def wait_scatter(b):
        pltpu.make_async_copy(zeros.at[pl.ds(0, K)], rows_v.at[b],
                              ssems[b]).wait()

    pltpu.sync_copy(zeros.at[pl.ds(sidx * RPT, RPT)],
                    acc_s.at[pl.ds(sidx * RPT, RPT)])

    plsc.subcore_barrier()

    stage(0)
    start_gather(0, 0, 0)

    def super_body(sup, carry):
        for ch in range(SCH):
            b = ch % 2
            wait_gather(b)
            if ch < SCH - 1:
                if ch == 0:
                    @pl.when(sup != 0)
                    def _():
                        wait_scatter(1 - b)
                else:
                    wait_scatter(1 - b)
                start_gather(sup, ch + 1, 1 - b)
                scale(ch, b)
                start_scatter(ch, b)
            else:
                scale(ch, b)
                start_scatter(ch, b)

                @pl.when(sup != nsup - 1)
                def _():
                    stage(sup + 1)
                    wait_scatter(1 - b)
                    start_gather(sup + 1, 0, 1 - b)
        return carry
    lax.fori_loop(0, nsup, super_body, 0)

    wait_scatter(0)
    wait_scatter(1)

    plsc.subcore_barrier()

    pltpu.sync_copy(acc_s.at[pl.ds(sidx * RPT, RPT)],
                    acc_out.at[cidx, pl.ds(sidx * RPT, RPT)])


def _make_sc_agg():
    mesh = plsc.VectorSubcoreMesh(core_axis_name="c", subcore_axis_name="s")
    scratch = [
        pltpu.VMEM((SCH, K), jnp.int32),
        pltpu.VMEM((SCH, K), jnp.int32),
        pltpu.VMEM((SCH, K), jnp.float32),
        pltpu.VMEM((2, K, D), jnp.float32),
        pltpu.VMEM_SHARED((NP, D), jnp.float32),
        pltpu.SemaphoreType.DMA,
        pltpu.SemaphoreType.DMA,
        pltpu.SemaphoreType.DMA,
        pltpu.SemaphoreType.DMA,
    ]
    return pl.kernel(
        functools.partial(_sc_agg_body, False),
        out_type=jax.ShapeDtypeStruct((NC, NP, D), jnp.float32),
        mesh=mesh,
        scratch_types=tuple(scratch),
    )


_sc_agg = _make_sc_agg()


def _sc_cnt_body(dst3, ones_h, zeros_c, cnt_out, dst_v, ones_v, cnt_s):
    cidx = lax.axis_index("c")
    sidx = lax.axis_index("s")
    wid = cidx * NS + sidx

    pltpu.sync_copy(zeros_c.at[pl.ds(sidx * RPT, RPT)],
                    cnt_s.at[pl.ds(sidx * RPT, RPT)])
    pltpu.sync_copy(ones_h, ones_v)
    plsc.subcore_barrier()

    def super_body(sc, carry):
        pltpu.sync_copy(dst3.at[wid * NSC + sc], dst_v)

        def chunk_body(ch, c1):
            pltpu.sync_copy(ones_v, cnt_s.at[dst_v.at[ch]], add=True)
            return c1
        lax.fori_loop(0, SCH, chunk_body, 0)
        return carry
    lax.fori_loop(0, NSC, super_body, 0)

    plsc.subcore_barrier()
    pltpu.sync_copy(cnt_s.at[pl.ds(sidx * RPT, RPT)],
                    cnt_out.at[cidx, pl.ds(sidx * RPT, RPT)])


_sc_cnt = pl.kernel(
    _sc_cnt_body,
    out_type=jax.ShapeDtypeStruct((NC, NP, D), jnp.float32),
    mesh=plsc.VectorSubcoreMesh(core_axis_name="c", subcore_axis_name="s"),
    scratch_types=(
        pltpu.VMEM((SCH, K), jnp.int32),
        pltpu.VMEM((K, D), jnp.float32),
        pltpu.VMEM_SHARED((NP, D), jnp.float32),
    ),
)

BT = 400
GT = N // BT


def _tc1_body(x_ref, acc_ref, cnt_ref, w1t_ref, b1_ref, w2t_ref, b2_ref,
              g_ref, s2_ref):
    cnt = cnt_ref[0, :, 0:1] + cnt_ref[1, :, 0:1]
    hN = (acc_ref[0] + acc_ref[1]) / jnp.maximum(cnt, 1.0)
    z = (jnp.dot(x_ref[...], w1t_ref[:D], preferred_element_type=jnp.float32)
         + jnp.dot(hN, w1t_ref[D:], preferred_element_type=jnp.float32)
         + b1_ref[...])
    h = jnp.maximum(z, 0.0)
    g_ref[...] = jnp.dot(h, w2t_ref[D:], preferred_element_type=jnp.float32)
    s2_ref[...] = (jnp.dot(h, w2t_ref[:D], preferred_element_type=jnp.float32)
                   + b2_ref[...])


_tc1 = pl.pallas_call(
    _tc1_body,
    grid=(GT,),
    in_specs=[
        pl.BlockSpec((BT, D), lambda i: (i, 0)),
        pl.BlockSpec((NC, BT, D), lambda i: (0, i, 0)),
        pl.BlockSpec((NC, BT, D), lambda i: (0, i, 0)),
        pl.BlockSpec((2 * D, D), lambda i: (0, 0)),
        pl.BlockSpec((1, D), lambda i: (0, 0)),
        pl.BlockSpec((2 * D, D), lambda i: (0, 0)),
        pl.BlockSpec((1, D), lambda i: (0, 0)),
    ],
    out_specs=[
        pl.BlockSpec((BT, D), lambda i: (i, 0)),
        pl.BlockSpec((BT, D), lambda i: (i, 0)),
    ],
    out_shape=[
        jax.ShapeDtypeStruct((N, D), jnp.float32),
        jax.ShapeDtypeStruct((N, D), jnp.float32),
    ],
)


def _tc2_body(s2_ref, acc_ref, cnt_ref, o_ref):
    cnt = cnt_ref[0, :, 0:1] + cnt_ref[1, :, 0:1]
    o_ref[...] = s2_ref[...] + (acc_ref[0] + acc_ref[1]) / jnp.maximum(cnt, 1.0)


_tc2 = pl.pallas_call(
    _tc2_body,
    grid=(GT,),
    in_specs=[
        pl.BlockSpec((BT, D), lambda i: (i, 0)),
        pl.BlockSpec((NC, BT, D), lambda i: (0, i, 0)),
        pl.BlockSpec((NC, BT, D), lambda i: (0, i, 0)),
    ],
    out_specs=pl.BlockSpec((BT, D), lambda i: (i, 0)),
    out_shape=jax.ShapeDtypeStruct((N, D), jnp.float32),
)


def kernel(x, edge_index, w, W1, b1, W2, b2):
    ei = edge_index.astype(jnp.int32)
    pad = E2 - E
    src_p = jnp.concatenate([ei[0], jnp.zeros((pad,), jnp.int32)])
    dst_p = jnp.concatenate([ei[1], jnp.full((pad,), N, jnp.int32)])
    w_p = jnp.concatenate([w.reshape(E).astype(jnp.float32),
                           jnp.zeros((pad,), jnp.float32)])
    src3 = src_p.reshape(TSUP, SCH, K)
    dst3 = dst_p.reshape(TSUP, SCH, K)
    w3 = w_p.reshape(TSUP, SCH, K)
    zeros = jnp.zeros((NP, D), jnp.float32)
    ones_h = jnp.ones((K, D), jnp.float32)
    W1T = W1.T
    W2T = W2.T
    b1r = b1.reshape(1, D)
    b2r = b2.reshape(1, D)

    cntp = _sc_cnt(dst3, ones_h, zeros)
    acc1 = _sc_agg(x, src3, dst3, w3, zeros)
    g, s2 = _tc1(x, acc1, cntp, W1T, b1r, W2T, b2r)
    acc2 = _sc_agg(g, src3, dst3, w3, zeros)
    return _tc2(s2, acc2, cntp)
